# baseline (speedup 1.0000x reference)
import functools

import jax
import jax.numpy as jnp
from jax import lax
from jax.experimental import pallas as pl
from jax.experimental.pallas import tpu as pltpu
from jax.experimental.pallas import tpu_sc as plsc

_LANES = 16
_NCHUNK = 7


@functools.cache
def _build(n, tab_n):
    info = plsc.get_sparse_core_info()
    nc, ns = info.num_cores, info.num_subcores
    nw = nc * ns
    b_per_w = (n // (nw * 8)) * 8
    tail = n - nw * b_per_w
    assert tail % _LANES == 0 and b_per_w % _LANES == 0
    assert b_per_w % (_NCHUNK * _LANES) == 0
    cs = b_per_w // _NCHUNK

    mesh = plsc.VectorSubcoreMesh(core_axis_name="c", subcore_axis_name="s")

    @functools.partial(
        pl.kernel,
        mesh=mesh,
        out_type=jax.ShapeDtypeStruct((n,), jnp.float32),
        compiler_params=pltpu.CompilerParams(
            needs_layout_passes=False,
            disable_bounds_checks=True,
            skip_device_barrier=True,
            use_tc_tiling_on_sc=False,
        ),
        scratch_types=[
            pltpu.VMEM((tab_n,), jnp.float32),
            pltpu.VMEM((2, cs), jnp.int32),
            pltpu.VMEM((2, cs), jnp.float32),
            pltpu.VMEM((max(tail, _LANES),), jnp.int32),
            pltpu.VMEM((max(tail, _LANES),), jnp.float32),
            pltpu.SemaphoreType.DMA,
            pltpu.SemaphoreType.DMA,
            pltpu.SemaphoreType.DMA,
            pltpu.SemaphoreType.DMA,
            pltpu.SemaphoreType.DMA,
        ],
    )
    def atomref_gather(z_hbm, tab_hbm, out_hbm,
                       tab_v, z_v, g_v, zt_v, gt_v,
                       sem_t, sem_z0, sem_z1, sem_s0, sem_s1):
        wid = lax.axis_index("s") * nc + lax.axis_index("c")
        base = wid * b_per_w
        zsems = (sem_z0, sem_z1)
        ssems = (sem_s0, sem_s1)

        ct = pltpu.async_copy(tab_hbm, tab_v, sem_t)
        loads = [None] * _NCHUNK
        stores = [None] * _NCHUNK
        for k in range(min(2, _NCHUNK)):
            loads[k] = pltpu.async_copy(
                z_hbm.at[pl.ds(base + k * cs, cs)], z_v.at[k], zsems[k])
        ct.wait()

        for k in range(_NCHUNK):
            buf = k % 2
            loads[k].wait()
            if k >= 2:
                stores[k - 2].wait()

            @plsc.parallel_loop(0, cs, step=_LANES, unroll=8)
            def body(o):
                zi = z_v[buf, pl.ds(o, _LANES)]
                g_v[buf, pl.ds(o, _LANES)] = plsc.load_gather(tab_v, [zi])

            stores[k] = pltpu.async_copy(
                g_v.at[buf], out_hbm.at[pl.ds(base + k * cs, cs)], ssems[buf])
            if k + 2 < _NCHUNK:
                loads[k + 2] = pltpu.async_copy(
                    z_hbm.at[pl.ds(base + (k + 2) * cs, cs)], z_v.at[buf],
                    zsems[buf])

        if tail:
            tbase = nw * b_per_w

            @pl.when(wid == nw - 1)
            def _():
                pltpu.sync_copy(z_hbm.at[pl.ds(tbase, tail)], zt_v)

                @plsc.parallel_loop(0, tail, step=_LANES, unroll=4)
                def tbody(o):
                    zi = zt_v[pl.ds(o, _LANES)]
                    gt_v[pl.ds(o, _LANES)] = plsc.load_gather(tab_v, [zi])

                pltpu.sync_copy(gt_v, out_hbm.at[pl.ds(tbase, tail)])

        for k in range(max(0, _NCHUNK - 2), _NCHUNK):
            stores[k].wait()

    return atomref_gather


def kernel(x, z, pos, batch, atomref):
    n = x.shape[0]
    tab = atomref.reshape(-1)
    gathered = _build(n, tab.shape[0])(z, tab)
    return x + gathered.reshape(n, 1)


# speedup vs baseline: 1.0250x; 1.0250x over previous
import functools

import jax
import jax.numpy as jnp
from jax import lax
from jax.experimental import pallas as pl
from jax.experimental.pallas import tpu as pltpu
from jax.experimental.pallas import tpu_sc as plsc

_LANES = 16
_NCHUNK = 3


@functools.cache
def _build(n, tab_n):
    info = plsc.get_sparse_core_info()
    nc, ns = info.num_cores, info.num_subcores
    nw = nc * ns
    b_per_w = (n // (nw * 8)) * 8
    tail = n - nw * b_per_w
    assert tail % _LANES == 0 and b_per_w % _LANES == 0
    assert b_per_w % (_NCHUNK * _LANES) == 0
    cs = b_per_w // _NCHUNK

    mesh = plsc.VectorSubcoreMesh(core_axis_name="c", subcore_axis_name="s")

    @functools.partial(
        pl.kernel,
        mesh=mesh,
        out_type=jax.ShapeDtypeStruct((n,), jnp.float32),
        compiler_params=pltpu.CompilerParams(
            needs_layout_passes=False,
            disable_bounds_checks=True,
            skip_device_barrier=True,
            use_tc_tiling_on_sc=False,
        ),
        scratch_types=[
            pltpu.VMEM((tab_n,), jnp.float32),
            pltpu.VMEM((2, cs), jnp.int32),
            pltpu.VMEM((2, cs), jnp.float32),
            pltpu.VMEM((max(tail, _LANES),), jnp.int32),
            pltpu.VMEM((max(tail, _LANES),), jnp.float32),
            pltpu.SemaphoreType.DMA,
            pltpu.SemaphoreType.DMA,
            pltpu.SemaphoreType.DMA,
            pltpu.SemaphoreType.DMA,
            pltpu.SemaphoreType.DMA,
        ],
    )
    def atomref_gather(z_hbm, tab_hbm, out_hbm,
                       tab_v, z_v, g_v, zt_v, gt_v,
                       sem_t, sem_z0, sem_z1, sem_s0, sem_s1):
        wid = lax.axis_index("s") * nc + lax.axis_index("c")
        base = wid * b_per_w
        zsems = (sem_z0, sem_z1)
        ssems = (sem_s0, sem_s1)

        ct = pltpu.async_copy(tab_hbm, tab_v, sem_t)
        loads = [None] * _NCHUNK
        stores = [None] * _NCHUNK
        for k in range(min(2, _NCHUNK)):
            loads[k] = pltpu.async_copy(
                z_hbm.at[pl.ds(base + k * cs, cs)], z_v.at[k], zsems[k])
        ct.wait()

        for k in range(_NCHUNK):
            buf = k % 2
            loads[k].wait()
            if k >= 2:
                stores[k - 2].wait()

            @plsc.parallel_loop(0, cs, step=_LANES, unroll=16)
            def body(o):
                zi = z_v[buf, pl.ds(o, _LANES)]
                g_v[buf, pl.ds(o, _LANES)] = plsc.load_gather(tab_v, [zi])

            stores[k] = pltpu.async_copy(
                g_v.at[buf], out_hbm.at[pl.ds(base + k * cs, cs)], ssems[buf])
            if k + 2 < _NCHUNK:
                loads[k + 2] = pltpu.async_copy(
                    z_hbm.at[pl.ds(base + (k + 2) * cs, cs)], z_v.at[buf],
                    zsems[buf])

        if tail:
            tbase = nw * b_per_w

            @pl.when(wid == nw - 1)
            def _():
                pltpu.sync_copy(z_hbm.at[pl.ds(tbase, tail)], zt_v)

                @plsc.parallel_loop(0, tail, step=_LANES, unroll=4)
                def tbody(o):
                    zi = zt_v[pl.ds(o, _LANES)]
                    gt_v[pl.ds(o, _LANES)] = plsc.load_gather(tab_v, [zi])

                pltpu.sync_copy(gt_v, out_hbm.at[pl.ds(tbase, tail)])

        for k in range(max(0, _NCHUNK - 2), _NCHUNK):
            stores[k].wait()

    return atomref_gather


def kernel(x, z, pos, batch, atomref):
    n = x.shape[0]
    tab = atomref.reshape(-1)
    gathered = _build(n, tab.shape[0])(z, tab)
    return x + gathered.reshape(n, 1)


# speedup vs baseline: 1.0327x; 1.0075x over previous
import functools

import jax
import jax.numpy as jnp
from jax import lax
from jax.experimental import pallas as pl
from jax.experimental.pallas import tpu as pltpu
from jax.experimental.pallas import tpu_sc as plsc

_LANES = 16
_NCHUNK = 3


@functools.cache
def _build(n, tab_n):
    info = plsc.get_sparse_core_info()
    nc, ns = info.num_cores, info.num_subcores
    nw = nc * ns
    b_per_w = (n // (nw * 8)) * 8
    tail = n - nw * b_per_w
    assert tail % _LANES == 0 and b_per_w % _LANES == 0
    assert b_per_w % (_NCHUNK * _LANES) == 0
    cs = b_per_w // _NCHUNK

    mesh = plsc.VectorSubcoreMesh(core_axis_name="c", subcore_axis_name="s")

    @functools.partial(
        pl.kernel,
        mesh=mesh,
        out_type=jax.ShapeDtypeStruct((n,), jnp.float32),
        compiler_params=pltpu.CompilerParams(
            needs_layout_passes=False,
            disable_bounds_checks=True,
            skip_device_barrier=True,
            use_tc_tiling_on_sc=False,
        ),
        scratch_types=[
            pltpu.VMEM((tab_n,), jnp.float32),
            pltpu.VMEM((2, cs), jnp.int32),
            pltpu.VMEM((2, cs), jnp.float32),
            pltpu.VMEM((max(tail, _LANES),), jnp.int32),
            pltpu.VMEM((max(tail, _LANES),), jnp.float32),
            pltpu.SemaphoreType.DMA,
            pltpu.SemaphoreType.DMA,
            pltpu.SemaphoreType.DMA,
            pltpu.SemaphoreType.DMA,
            pltpu.SemaphoreType.DMA,
        ],
    )
    def atomref_gather(z_hbm, tab_hbm, out_hbm,
                       tab_v, z_v, g_v, zt_v, gt_v,
                       sem_t, sem_z0, sem_z1, sem_s0, sem_s1):
        wid = lax.axis_index("s") * nc + lax.axis_index("c")
        base = wid * b_per_w
        zsems = (sem_z0, sem_z1)
        ssems = (sem_s0, sem_s1)

        ct = pltpu.async_copy(tab_hbm, tab_v, sem_t)
        loads = [None] * _NCHUNK
        stores = [None] * _NCHUNK
        for k in range(min(2, _NCHUNK)):
            loads[k] = pltpu.async_copy(
                z_hbm.at[pl.ds(base + k * cs, cs)], z_v.at[k], zsems[k])
        ct.wait()

        for k in range(_NCHUNK):
            buf = k % 2
            loads[k].wait()
            if k >= 2:
                stores[k - 2].wait()

            @plsc.parallel_loop(0, cs, step=_LANES, unroll=8)
            def body(o):
                zi = z_v[buf, pl.ds(o, _LANES)]
                g_v[buf, pl.ds(o, _LANES)] = plsc.load_gather(tab_v, [zi])

            stores[k] = pltpu.async_copy(
                g_v.at[buf], out_hbm.at[pl.ds(base + k * cs, cs)], ssems[buf])
            if k + 2 < _NCHUNK:
                loads[k + 2] = pltpu.async_copy(
                    z_hbm.at[pl.ds(base + (k + 2) * cs, cs)], z_v.at[buf],
                    zsems[buf])

        if tail:
            tbase = nw * b_per_w

            @pl.when(wid == nw - 1)
            def _():
                pltpu.sync_copy(z_hbm.at[pl.ds(tbase, tail)], zt_v)

                @plsc.parallel_loop(0, tail, step=_LANES, unroll=4)
                def tbody(o):
                    zi = zt_v[pl.ds(o, _LANES)]
                    gt_v[pl.ds(o, _LANES)] = plsc.load_gather(tab_v, [zi])

                pltpu.sync_copy(gt_v, out_hbm.at[pl.ds(tbase, tail)])

        for k in range(max(0, _NCHUNK - 2), _NCHUNK):
            stores[k].wait()

    return atomref_gather


def kernel(x, z, pos, batch, atomref):
    n = x.shape[0]
    tab = atomref.reshape(-1)
    gathered = _build(n, tab.shape[0])(z, tab)
    return x + gathered.reshape(n, 1)


# speedup vs baseline: 1.0333x; 1.0006x over previous
import functools

import jax
import jax.numpy as jnp
from jax import lax
from jax.experimental import pallas as pl
from jax.experimental.pallas import tpu as pltpu
from jax.experimental.pallas import tpu_sc as plsc

_LANES = 16
_NCHUNK = 3


@functools.cache
def _build(n, tab_n):
    info = plsc.get_sparse_core_info()
    nc, ns = info.num_cores, info.num_subcores
    nw = nc * ns
    b_per_w = (n // (nw * 8)) * 8
    tail = n - nw * b_per_w
    assert tail % _LANES == 0 and b_per_w % _LANES == 0
    assert b_per_w % (_NCHUNK * _LANES) == 0
    cs = b_per_w // _NCHUNK

    mesh = plsc.VectorSubcoreMesh(core_axis_name="c", subcore_axis_name="s")

    @functools.partial(
        pl.kernel,
        mesh=mesh,
        out_type=jax.ShapeDtypeStruct((n,), jnp.float32),
        compiler_params=pltpu.CompilerParams(
            needs_layout_passes=False,
            disable_bounds_checks=True,
            skip_device_barrier=True,
            use_tc_tiling_on_sc=False,
        ),
        scratch_types=[
            pltpu.VMEM((tab_n,), jnp.float32),
            pltpu.VMEM((2, cs), jnp.int32),
            pltpu.VMEM((2, cs), jnp.float32),
            pltpu.VMEM((max(tail, _LANES),), jnp.int32),
            pltpu.VMEM((max(tail, _LANES),), jnp.float32),
            pltpu.SemaphoreType.DMA,
            pltpu.SemaphoreType.DMA,
            pltpu.SemaphoreType.DMA,
            pltpu.SemaphoreType.DMA,
            pltpu.SemaphoreType.DMA,
        ],
    )
    def atomref_gather(z_hbm, tab_hbm, out_hbm,
                       tab_v, z_v, g_v, zt_v, gt_v,
                       sem_t, sem_z0, sem_z1, sem_s0, sem_s1):
        wid = lax.axis_index("s") * nc + lax.axis_index("c")
        base = wid * b_per_w
        zsems = (sem_z0, sem_z1)
        ssems = (sem_s0, sem_s1)

        ct = pltpu.async_copy(tab_hbm, tab_v, sem_t)
        loads = [None] * _NCHUNK
        stores = [None] * _NCHUNK
        for k in range(min(2, _NCHUNK)):
            loads[k] = pltpu.async_copy(
                z_hbm.at[pl.ds(base + k * cs, cs)], z_v.at[k], zsems[k])
        ct.wait()

        if tail:
            tbase = nw * b_per_w

            @pl.when(wid == nw - 1)
            def _():
                pltpu.async_copy(z_hbm.at[pl.ds(tbase, tail)], zt_v, sem_t)

        for k in range(_NCHUNK):
            buf = k % 2
            loads[k].wait()
            if k >= 2:
                stores[k - 2].wait()

            @plsc.parallel_loop(0, cs, step=_LANES, unroll=8)
            def body(o):
                zi = z_v[buf, pl.ds(o, _LANES)]
                g_v[buf, pl.ds(o, _LANES)] = plsc.load_gather(tab_v, [zi])

            stores[k] = pltpu.async_copy(
                g_v.at[buf], out_hbm.at[pl.ds(base + k * cs, cs)], ssems[buf])
            if k + 2 < _NCHUNK:
                loads[k + 2] = pltpu.async_copy(
                    z_hbm.at[pl.ds(base + (k + 2) * cs, cs)], z_v.at[buf],
                    zsems[buf])

        if tail:
            @pl.when(wid == nw - 1)
            def _():
                pltpu.make_async_copy(
                    z_hbm.at[pl.ds(tbase, tail)], zt_v, sem_t).wait()

                @plsc.parallel_loop(0, tail, step=_LANES, unroll=4)
                def tbody(o):
                    zi = zt_v[pl.ds(o, _LANES)]
                    gt_v[pl.ds(o, _LANES)] = plsc.load_gather(tab_v, [zi])

                pltpu.sync_copy(gt_v, out_hbm.at[pl.ds(tbase, tail)])

        for k in range(max(0, _NCHUNK - 2), _NCHUNK):
            stores[k].wait()

    return atomref_gather


def kernel(x, z, pos, batch, atomref):
    n = x.shape[0]
    tab = atomref.reshape(-1)
    gathered = _build(n, tab.shape[0])(z, tab)
    return x + gathered.reshape(n, 1)
